# baseline (speedup 1.0000x reference)
import numpy as np

import jax
import jax.numpy as jnp
from jax import lax
from jax.experimental import pallas as pl

STRIDE = 8
IMG_H = 512
IMG_W = 512
NA = 9
NCLS = 80
NB = 8
NH = IMG_H // STRIDE
NW = IMG_W // STRIDE
ROWS = NH * NW


def _anchor_wh_np():
    base = 4 * STRIDE
    scales = [2.0 ** 0.0, 2.0 ** (1.0 / 3.0), 2.0 ** (2.0 / 3.0)]
    ratios = [(1.0, 1.0), (1.4, 0.7), (0.7, 1.4)]
    anchors = [(base * sc * rt[0], base * sc * rt[1]) for sc in scales for rt in ratios]
    return np.array(anchors, dtype=np.float32)


def _retina_body(awh_ref, t_ref, cls_ref, bbox_ref, idx_ref, score_ref):
    aw = awh_ref[0, 0, 0]
    ah = awh_ref[0, 0, 1]

    t = t_ref[0, 0]
    rows = lax.broadcasted_iota(jnp.int32, (ROWS, 1), 0)
    a_cx = (rows % NW).astype(jnp.float32) * STRIDE + STRIDE / 2
    a_cy = (rows // NW).astype(jnp.float32) * STRIDE + STRIDE / 2
    px = a_cx + t[:, 0:1] * aw
    py = a_cy + t[:, 1:2] * ah
    pw = jnp.exp(t[:, 2:3]) * aw
    ph = jnp.exp(t[:, 3:4]) * ah
    bbox = jnp.concatenate([px, py, pw, ph], axis=1)
    bbox_ref[0, 0] = jnp.clip(bbox, 1.0, float(max(IMG_H, IMG_W)))

    x = cls_ref[0, 0]
    m = jnp.max(x, axis=1, keepdims=True)
    li = lax.broadcasted_iota(jnp.int32, (ROWS, NCLS), 1)
    idx = jnp.min(jnp.where(x == m, li, NCLS), axis=1, keepdims=True)
    idx_ref[0, 0] = idx
    score_ref[0, 0] = jax.nn.sigmoid(m)


def kernel(t_xywh, cls_logits):
    t = t_xywh.reshape(NB, NA, ROWS, 4)
    cls = cls_logits.reshape(NB, NA, ROWS, NCLS)
    awh = jnp.asarray(_anchor_wh_np()).reshape(NA, 1, 2)

    bbox, idx, score = pl.pallas_call(
        _retina_body,
        grid=(NB, NA),
        in_specs=[
            pl.BlockSpec((1, 1, 2), lambda b, a: (a, 0, 0)),
            pl.BlockSpec((1, 1, ROWS, 4), lambda b, a: (b, a, 0, 0)),
            pl.BlockSpec((1, 1, ROWS, NCLS), lambda b, a: (b, a, 0, 0)),
        ],
        out_specs=[
            pl.BlockSpec((1, 1, ROWS, 4), lambda b, a: (b, a, 0, 0)),
            pl.BlockSpec((1, 1, ROWS, 1), lambda b, a: (b, a, 0, 0)),
            pl.BlockSpec((1, 1, ROWS, 1), lambda b, a: (b, a, 0, 0)),
        ],
        out_shape=[
            jax.ShapeDtypeStruct((NB, NA, ROWS, 4), jnp.float32),
            jax.ShapeDtypeStruct((NB, NA, ROWS, 1), jnp.int32),
            jax.ShapeDtypeStruct((NB, NA, ROWS, 1), jnp.float32),
        ],
    )(awh, t, cls)

    return (
        bbox.reshape(NB, NA * ROWS, 4),
        idx.reshape(NB, NA * ROWS),
        score.reshape(NB, NA * ROWS),
    )


# speedup vs baseline: 1.3399x; 1.3399x over previous
import numpy as np

import jax
import jax.numpy as jnp
from jax import lax
from jax.experimental import pallas as pl

STRIDE = 8
IMG_H = 512
IMG_W = 512
NA = 9
NCLS = 80
NB = 8
NH = IMG_H // STRIDE
NW = IMG_W // STRIDE
ROWS = NH * NW


def _anchor_wh_np():
    base = 4 * STRIDE
    scales = [2.0 ** 0.0, 2.0 ** (1.0 / 3.0), 2.0 ** (2.0 / 3.0)]
    ratios = [(1.0, 1.0), (1.4, 0.7), (0.7, 1.4)]
    anchors = [(base * sc * rt[0], base * sc * rt[1]) for sc in scales for rt in ratios]
    return np.array(anchors, dtype=np.float32)


def _retina_body(awh_ref, t_ref, cls_ref, bbox_ref, idx_ref, score_ref):
    aw = awh_ref[0, 0, 0]
    ah = awh_ref[0, 0, 1]

    tt = t_ref[0, 0]
    r = lax.broadcasted_iota(jnp.int32, (128, 128), 0)
    c = lax.broadcasted_iota(jnp.int32, (128, 128), 1)
    row = r * 32 + (c >> 2)
    comp = c & 3
    wf = (row & (NW - 1)).astype(jnp.float32)
    hf = (row >> 6).astype(jnp.float32)
    scale = jnp.where((c & 1) == 0, aw, ah)
    off = jnp.where(comp == 0, wf * STRIDE + STRIDE / 2,
                    jnp.where(comp == 1, hf * STRIDE + STRIDE / 2, 0.0))
    val = jnp.where(comp >= 2, jnp.exp(tt) * scale, off + tt * scale)
    bbox_ref[0, 0] = jnp.clip(val, 1.0, float(max(IMG_H, IMG_W)))

    x = cls_ref[0, 0]
    m = jnp.max(x, axis=1, keepdims=True)
    rev = lax.broadcasted_iota(jnp.int32, (ROWS, NCLS), 1).astype(jnp.float32)
    picked = jnp.max(jnp.where(x == m, (NCLS - 1.0) - rev, -1.0),
                     axis=1, keepdims=True)
    idx_ref[0, 0] = ((NCLS - 1.0) - picked).astype(jnp.int32)
    score_ref[0, 0] = jax.nn.sigmoid(m)


def kernel(t_xywh, cls_logits):
    t = t_xywh.reshape(NB, NA, 128, 128)
    cls = cls_logits.reshape(NB, NA, ROWS, NCLS)
    awh = jnp.asarray(_anchor_wh_np()).reshape(NA, 1, 2)

    bbox, idx, score = pl.pallas_call(
        _retina_body,
        grid=(NB, NA),
        in_specs=[
            pl.BlockSpec((1, 1, 2), lambda b, a: (a, 0, 0)),
            pl.BlockSpec((1, 1, 128, 128), lambda b, a: (b, a, 0, 0)),
            pl.BlockSpec((1, 1, ROWS, NCLS), lambda b, a: (b, a, 0, 0)),
        ],
        out_specs=[
            pl.BlockSpec((1, 1, 128, 128), lambda b, a: (b, a, 0, 0)),
            pl.BlockSpec((1, 1, ROWS, 1), lambda b, a: (b, a, 0, 0)),
            pl.BlockSpec((1, 1, ROWS, 1), lambda b, a: (b, a, 0, 0)),
        ],
        out_shape=[
            jax.ShapeDtypeStruct((NB, NA, 128, 128), jnp.float32),
            jax.ShapeDtypeStruct((NB, NA, ROWS, 1), jnp.int32),
            jax.ShapeDtypeStruct((NB, NA, ROWS, 1), jnp.float32),
        ],
    )(awh, t, cls)

    return (
        bbox.reshape(NB, NA * ROWS, 4),
        idx.reshape(NB, NA * ROWS),
        score.reshape(NB, NA * ROWS),
    )


# speedup vs baseline: 2.0163x; 1.5048x over previous
import numpy as np

import jax
import jax.numpy as jnp
from jax import lax
from jax.experimental import pallas as pl

STRIDE = 8
IMG_H = 512
IMG_W = 512
NA = 9
NCLS = 80
NB = 8
NH = IMG_H // STRIDE
NW = IMG_W // STRIDE
ROWS = NH * NW


def _anchor_wh_np():
    base = 4 * STRIDE
    scales = [2.0 ** 0.0, 2.0 ** (1.0 / 3.0), 2.0 ** (2.0 / 3.0)]
    ratios = [(1.0, 1.0), (1.4, 0.7), (0.7, 1.4)]
    anchors = [(base * sc * rt[0], base * sc * rt[1]) for sc in scales for rt in ratios]
    return np.array(anchors, dtype=np.float32)


def _retina_body(awh_ref, t_ref, cls_ref, bbox_ref, idx_ref, score_ref):
    aw = awh_ref[0, 0, 0]
    ah = awh_ref[0, 0, 1]

    tt = t_ref[0, 0]
    r = lax.broadcasted_iota(jnp.int32, (128, 128), 0)
    c = lax.broadcasted_iota(jnp.int32, (128, 128), 1)
    row = r * 32 + (c >> 2)
    comp = c & 3
    wf = (row & (NW - 1)).astype(jnp.float32)
    hf = (row >> 6).astype(jnp.float32)
    scale = jnp.where((c & 1) == 0, aw, ah)
    off = jnp.where(comp == 0, wf * STRIDE + STRIDE / 2,
                    jnp.where(comp == 1, hf * STRIDE + STRIDE / 2, 0.0))
    val = jnp.where(comp >= 2, jnp.exp(tt) * scale, off + tt * scale)
    bbox_ref[0, 0] = jnp.clip(val, 1.0, float(max(IMG_H, IMG_W)))

    x = cls_ref[0, 0]
    m = jnp.max(x, axis=1, keepdims=True)
    rev = lax.broadcasted_iota(jnp.int32, (ROWS, NCLS), 1).astype(jnp.float32)
    picked = jnp.max(jnp.where(x == m, (NCLS - 1.0) - rev, -1.0),
                     axis=1, keepdims=True)
    picked2 = picked.reshape(32, 128)
    m2 = m.reshape(32, 128)
    idx_ref[0, 0] = ((NCLS - 1.0) - picked2).astype(jnp.int32)
    score_ref[0, 0] = jax.nn.sigmoid(m2)


def kernel(t_xywh, cls_logits):
    t = t_xywh.reshape(NB, NA, 128, 128)
    cls = cls_logits.reshape(NB, NA, ROWS, NCLS)
    awh = jnp.asarray(_anchor_wh_np()).reshape(NA, 1, 2)

    bbox, idx, score = pl.pallas_call(
        _retina_body,
        grid=(NB, NA),
        in_specs=[
            pl.BlockSpec((1, 1, 2), lambda b, a: (a, 0, 0)),
            pl.BlockSpec((1, 1, 128, 128), lambda b, a: (b, a, 0, 0)),
            pl.BlockSpec((1, 1, ROWS, NCLS), lambda b, a: (b, a, 0, 0)),
        ],
        out_specs=[
            pl.BlockSpec((1, 1, 128, 128), lambda b, a: (b, a, 0, 0)),
            pl.BlockSpec((1, 1, 32, 128), lambda b, a: (b, a, 0, 0)),
            pl.BlockSpec((1, 1, 32, 128), lambda b, a: (b, a, 0, 0)),
        ],
        out_shape=[
            jax.ShapeDtypeStruct((NB, NA, 128, 128), jnp.float32),
            jax.ShapeDtypeStruct((NB, NA, 32, 128), jnp.int32),
            jax.ShapeDtypeStruct((NB, NA, 32, 128), jnp.float32),
        ],
    )(awh, t, cls)

    return (
        bbox.reshape(NB, NA * ROWS, 4),
        idx.reshape(NB, NA * ROWS),
        score.reshape(NB, NA * ROWS),
    )


# speedup vs baseline: 2.0500x; 1.0167x over previous
import numpy as np

import jax
import jax.numpy as jnp
from jax import lax
from jax.experimental import pallas as pl

STRIDE = 8
IMG_H = 512
IMG_W = 512
NA = 9
NCLS = 80
NB = 8
NH = IMG_H // STRIDE
NW = IMG_W // STRIDE
ROWS = NH * NW
NSPLIT = 4
CHUNK = ROWS // NSPLIT


def _anchor_wh_np():
    base = 4 * STRIDE
    scales = [2.0 ** 0.0, 2.0 ** (1.0 / 3.0), 2.0 ** (2.0 / 3.0)]
    ratios = [(1.0, 1.0), (1.4, 0.7), (0.7, 1.4)]
    anchors = [(base * sc * rt[0], base * sc * rt[1]) for sc in scales for rt in ratios]
    return np.array(anchors, dtype=np.float32)


def _cls_reduce(x):
    m = jnp.max(x, axis=1, keepdims=True)
    rev = lax.broadcasted_iota(jnp.int32, (CHUNK, NCLS), 1).astype(jnp.float32)
    picked = jnp.max(jnp.where(x == m, (NCLS - 1.0) - rev, -1.0),
                     axis=1, keepdims=True)
    return m.reshape(CHUNK // 128, 128), picked.reshape(CHUNK // 128, 128)


def _retina_body(awh_ref, t_ref, c0_ref, c1_ref, c2_ref, c3_ref,
                 bbox_ref, idx_ref, score_ref):
    aw = awh_ref[0, 0, 0]
    ah = awh_ref[0, 0, 1]

    tt = t_ref[0, 0]
    r = lax.broadcasted_iota(jnp.int32, (128, 128), 0)
    c = lax.broadcasted_iota(jnp.int32, (128, 128), 1)
    row = r * 32 + (c >> 2)
    comp = c & 3
    wf = (row & (NW - 1)).astype(jnp.float32)
    hf = (row >> 6).astype(jnp.float32)
    scale = jnp.where((c & 1) == 0, aw, ah)
    off = jnp.where(comp == 0, wf * STRIDE + STRIDE / 2,
                    jnp.where(comp == 1, hf * STRIDE + STRIDE / 2, 0.0))
    val = jnp.where(comp >= 2, jnp.exp(tt) * scale, off + tt * scale)
    bbox_ref[0, 0] = jnp.clip(val, 1.0, float(max(IMG_H, IMG_W)))

    parts = [_cls_reduce(ref[0, 0]) for ref in (c0_ref, c1_ref, c2_ref, c3_ref)]
    m2 = jnp.concatenate([p[0] for p in parts], axis=0)
    picked2 = jnp.concatenate([p[1] for p in parts], axis=0)
    idx_ref[0, 0] = ((NCLS - 1.0) - picked2).astype(jnp.int32)
    score_ref[0, 0] = jax.nn.sigmoid(m2)


def kernel(t_xywh, cls_logits):
    t = t_xywh.reshape(NB, NA, 128, 128)
    cls = cls_logits.reshape(NB, NA, ROWS, NCLS)
    awh = jnp.asarray(_anchor_wh_np()).reshape(NA, 1, 2)

    def _cls_spec(k):
        return pl.BlockSpec((1, 1, CHUNK, NCLS), lambda b, a, k=k: (b, a, k, 0))

    bbox, idx, score = pl.pallas_call(
        _retina_body,
        grid=(NB, NA),
        in_specs=[
            pl.BlockSpec((1, 1, 2), lambda b, a: (a, 0, 0)),
            pl.BlockSpec((1, 1, 128, 128), lambda b, a: (b, a, 0, 0)),
            _cls_spec(0), _cls_spec(1), _cls_spec(2), _cls_spec(3),
        ],
        out_specs=[
            pl.BlockSpec((1, 1, 128, 128), lambda b, a: (b, a, 0, 0)),
            pl.BlockSpec((1, 1, 32, 128), lambda b, a: (b, a, 0, 0)),
            pl.BlockSpec((1, 1, 32, 128), lambda b, a: (b, a, 0, 0)),
        ],
        out_shape=[
            jax.ShapeDtypeStruct((NB, NA, 128, 128), jnp.float32),
            jax.ShapeDtypeStruct((NB, NA, 32, 128), jnp.int32),
            jax.ShapeDtypeStruct((NB, NA, 32, 128), jnp.float32),
        ],
    )(awh, t, cls, cls, cls, cls)

    return (
        bbox.reshape(NB, NA * ROWS, 4),
        idx.reshape(NB, NA * ROWS),
        score.reshape(NB, NA * ROWS),
    )


# speedup vs baseline: 2.2030x; 1.0746x over previous
import numpy as np

import jax
import jax.numpy as jnp
from jax import lax
from jax.experimental import pallas as pl

STRIDE = 8
IMG_H = 512
IMG_W = 512
NA = 9
NCLS = 80
NB = 8
NH = IMG_H // STRIDE
NW = IMG_W // STRIDE
ROWS = NH * NW
NSPLIT = 4
CHUNK = ROWS // NSPLIT
GA = 3


def _scale_tab_np():
    base = 4 * STRIDE
    scales = [2.0 ** 0.0, 2.0 ** (1.0 / 3.0), 2.0 ** (2.0 / 3.0)]
    ratios = [(1.0, 1.0), (1.4, 0.7), (0.7, 1.4)]
    anchors = [(base * sc * rt[0], base * sc * rt[1]) for sc in scales for rt in ratios]
    awh = np.array(anchors, dtype=np.float32)
    tab = np.empty((NA, 128), dtype=np.float32)
    tab[:, 0::2] = awh[:, 0:1]
    tab[:, 1::2] = awh[:, 1:2]
    return tab


def _cls_reduce(x):
    x2 = x.reshape(GA * CHUNK, NCLS)
    m = jnp.max(x2, axis=1, keepdims=True)
    rev = lax.broadcasted_iota(jnp.int32, (GA * CHUNK, NCLS), 1).astype(jnp.float32)
    picked = jnp.max(jnp.where(x2 == m, (NCLS - 1.0) - rev, -1.0),
                     axis=1, keepdims=True)
    return (m.reshape(GA, CHUNK // 128, 128),
            picked.reshape(GA, CHUNK // 128, 128))


def _retina_body(tab_ref, t_ref, c0_ref, c1_ref, c2_ref, c3_ref,
                 bbox_ref, idx_ref, score_ref):
    tt = t_ref[0]
    shp = (GA, 128, 128)
    r = lax.broadcasted_iota(jnp.int32, shp, 1)
    c = lax.broadcasted_iota(jnp.int32, shp, 2)
    row = r * 32 + (c >> 2)
    comp = c & 3
    wf = (row & (NW - 1)).astype(jnp.float32)
    hf = (row >> 6).astype(jnp.float32)
    scale = tab_ref[0][:, None, :]
    off = jnp.where(comp == 0, wf * STRIDE + STRIDE / 2,
                    jnp.where(comp == 1, hf * STRIDE + STRIDE / 2, 0.0))
    val = jnp.where(comp >= 2, jnp.exp(tt) * scale, off + tt * scale)
    bbox_ref[0] = jnp.clip(val, 1.0, float(max(IMG_H, IMG_W)))

    parts = [_cls_reduce(ref[0]) for ref in (c0_ref, c1_ref, c2_ref, c3_ref)]
    m2 = jnp.concatenate([p[0] for p in parts], axis=1)
    picked2 = jnp.concatenate([p[1] for p in parts], axis=1)
    idx_ref[0] = ((NCLS - 1.0) - picked2).astype(jnp.int32)
    score_ref[0] = jax.nn.sigmoid(m2)


def kernel(t_xywh, cls_logits):
    t = t_xywh.reshape(NB, NA, 128, 128)
    cls = cls_logits.reshape(NB, NA, ROWS, NCLS)
    tab = jnp.asarray(_scale_tab_np()).reshape(NA // GA, GA, 128)

    def _cls_spec(k):
        return pl.BlockSpec((1, GA, CHUNK, NCLS), lambda b, g, k=k: (b, g, k, 0))

    bbox, idx, score = pl.pallas_call(
        _retina_body,
        grid=(NB, NA // GA),
        in_specs=[
            pl.BlockSpec((1, GA, 128), lambda b, g: (g, 0, 0)),
            pl.BlockSpec((1, GA, 128, 128), lambda b, g: (b, g, 0, 0)),
            _cls_spec(0), _cls_spec(1), _cls_spec(2), _cls_spec(3),
        ],
        out_specs=[
            pl.BlockSpec((1, GA, 128, 128), lambda b, g: (b, g, 0, 0)),
            pl.BlockSpec((1, GA, 32, 128), lambda b, g: (b, g, 0, 0)),
            pl.BlockSpec((1, GA, 32, 128), lambda b, g: (b, g, 0, 0)),
        ],
        out_shape=[
            jax.ShapeDtypeStruct((NB, NA, 128, 128), jnp.float32),
            jax.ShapeDtypeStruct((NB, NA, 32, 128), jnp.int32),
            jax.ShapeDtypeStruct((NB, NA, 32, 128), jnp.float32),
        ],
    )(tab, t, cls, cls, cls, cls)

    return (
        bbox.reshape(NB, NA * ROWS, 4),
        idx.reshape(NB, NA * ROWS),
        score.reshape(NB, NA * ROWS),
    )
